# Initial kernel scaffold; baseline (speedup 1.0000x reference)
import jax
import jax.numpy as jnp
from jax.experimental import pallas as pl


def kernel(query_features, key_features, query_positions, key_positions, Wq, bq, Wk, bk, Wv, bv, Wo, bo):
    raise NotImplementedError("write your pallas kernel here")


# speedup vs baseline: 1.1368x; 1.1368x over previous
import functools

import jax
import jax.numpy as jnp
from jax import lax
from jax.experimental import pallas as pl
from jax.experimental.pallas import tpu as pltpu
from jax.experimental.pallas import tpu_sc as plsc

N1 = 10000
N2 = 10000
NPAD = 10240
D = 512
H = 8
DH = 64
K = 16
SCALE = DH ** -0.5

BQ = 200
CH = 128
BN = 80
BP = 1000


def _knn_body(qp_ref, kpt_ref, idx_ref, d2_ref):
    qp = qp_ref[...]
    kpt = kpt_ref[...]
    q2 = qp[:, 3:4]
    k2 = kpt[4:5, :]
    qk = lax.dot_general(qp, kpt, (((1,), (0,)), ((), ())),
                         preferred_element_type=jnp.float32)
    col = lax.broadcasted_iota(jnp.int32, (1, NPAD), 1)
    dist = jnp.sqrt(jnp.maximum((q2 + k2) - 2.0 * qk, 0.0))
    d2_ref[...] = jnp.where(col < N2, dist, jnp.inf)

    lane = lax.broadcasted_iota(jnp.int32, (BQ, CH), 1)
    l0 = jnp.full((BQ, K), jnp.inf, jnp.float32)
    i0 = jnp.zeros((BQ, K), jnp.int32)

    def chunk_step(c, carry):
        lbuf, ibuf = carry
        chunk = d2_ref[:, pl.ds(c * CH, CH)]

        def cond(st):
            ch, lb, _ = st
            return jnp.any(jnp.min(ch, axis=1, keepdims=True) < lb[:, K - 1:K])

        def body(st):
            ch, lb, ib = st
            x = jnp.min(ch, axis=1, keepdims=True)
            alane = jnp.min(jnp.where(ch == x, lane, CH), axis=1,
                            keepdims=True)
            gidx = c * CH + alane
            ch = jnp.where(lane == alane, jnp.inf, ch)
            lsh = jnp.concatenate(
                [jnp.full((BQ, 1), -jnp.inf, jnp.float32), lb[:, :K - 1]], 1)
            ish = jnp.concatenate(
                [jnp.zeros((BQ, 1), jnp.int32), ib[:, :K - 1]], 1)
            lnew = jnp.maximum(jnp.minimum(lb, x), lsh)
            inew = jnp.where(lb <= x, ib,
                             jnp.where(lsh <= x,
                                       jnp.broadcast_to(gidx, (BQ, K)), ish))
            return ch, lnew, inew

        _, lbuf, ibuf = lax.while_loop(cond, body, (chunk, lbuf, ibuf))
        return lbuf, ibuf

    _, ibuf = lax.fori_loop(0, NPAD // CH, chunk_step, (l0, i0))
    idx_ref[...] = ibuf


def _knn(qp_pad, kpt_pad):
    return pl.pallas_call(
        _knn_body,
        grid=(N1 // BQ,),
        in_specs=[
            pl.BlockSpec((BQ, 8), lambda i: (i, 0)),
            pl.BlockSpec((8, NPAD), lambda i: (0, 0)),
        ],
        out_specs=pl.BlockSpec((BQ, K), lambda i: (i, 0)),
        out_shape=jax.ShapeDtypeStruct((N1, K), jnp.int32),
        scratch_shapes=[pltpu.VMEM((BQ, NPAD), jnp.float32)],
    )(qp_pad, kpt_pad)


def _proj_body(x_ref, wt_ref, b_ref, o_ref):
    o_ref[...] = (jnp.dot(x_ref[...], wt_ref[...],
                          preferred_element_type=jnp.float32)
                  + b_ref[0:1, :])


def _proj(x, wt, b8):
    n = x.shape[0]
    return pl.pallas_call(
        _proj_body,
        grid=(n // BP,),
        in_specs=[
            pl.BlockSpec((BP, D), lambda i: (i, 0)),
            pl.BlockSpec((D, D), lambda i: (0, 0)),
            pl.BlockSpec((8, D), lambda i: (0, 0)),
        ],
        out_specs=pl.BlockSpec((BP, D), lambda i: (i, 0)),
        out_shape=jax.ShapeDtypeStruct((n, D), jnp.float32),
    )(x, wt, b8)


def _proj2_body(x_ref, wt1_ref, b1_ref, wt2_ref, b2_ref, o1_ref, o2_ref):
    x = x_ref[...]
    o1_ref[...] = (jnp.dot(x, wt1_ref[...],
                           preferred_element_type=jnp.float32) + b1_ref[0:1, :])
    o2_ref[...] = (jnp.dot(x, wt2_ref[...],
                           preferred_element_type=jnp.float32) + b2_ref[0:1, :])


def _proj2(x, wt1, b18, wt2, b28):
    n = x.shape[0]
    return pl.pallas_call(
        _proj2_body,
        grid=(n // BP,),
        in_specs=[
            pl.BlockSpec((BP, D), lambda i: (i, 0)),
            pl.BlockSpec((D, D), lambda i: (0, 0)),
            pl.BlockSpec((8, D), lambda i: (0, 0)),
            pl.BlockSpec((D, D), lambda i: (0, 0)),
            pl.BlockSpec((8, D), lambda i: (0, 0)),
        ],
        out_specs=[
            pl.BlockSpec((BP, D), lambda i: (i, 0)),
            pl.BlockSpec((BP, D), lambda i: (i, 0)),
        ],
        out_shape=[
            jax.ShapeDtypeStruct((n, D), jnp.float32),
            jax.ShapeDtypeStruct((n, D), jnp.float32),
        ],
    )(x, wt1, b18, wt2, b28)


GC = 40


def _gather(ktab, vtab, idx_flat):
    info = plsc.get_sparse_core_info()
    nw = info.num_cores * info.num_subcores
    b_per_w = (N1 * K) // nw
    nchunks = b_per_w // GC
    mesh = plsc.VectorSubcoreMesh(core_axis_name="c", subcore_axis_name="s")

    @functools.partial(
        pl.kernel, mesh=mesh,
        out_type=[
            jax.ShapeDtypeStruct((N1 * K, D), jnp.float32),
            jax.ShapeDtypeStruct((N1 * K, D), jnp.float32),
        ],
        scratch_types=[
            pltpu.VMEM((b_per_w,), jnp.int32),
            pltpu.VMEM((GC, D), jnp.float32),
            pltpu.VMEM((GC, D), jnp.float32),
            pltpu.SemaphoreType.DMA,
            pltpu.SemaphoreType.DMA,
        ],
    )
    def body(k_hbm, v_hbm, idx_hbm, knb_hbm, vnb_hbm,
             idx_v, krows, vrows, ksem, vsem):
        wid = lax.axis_index("s") * info.num_cores + lax.axis_index("c")
        base = wid * b_per_w
        pltpu.sync_copy(idx_hbm.at[pl.ds(base, b_per_w)], idx_v)

        def chunk(j, carry):
            off = j * GC
            isl = idx_v.at[pl.ds(off, GC)]
            pltpu.async_copy(k_hbm.at[isl], krows, ksem).wait()
            pltpu.async_copy(v_hbm.at[isl], vrows, vsem).wait()
            pltpu.sync_copy(krows, knb_hbm.at[pl.ds(base + off, GC)])
            pltpu.sync_copy(vrows, vnb_hbm.at[pl.ds(base + off, GC)])
            return carry

        lax.fori_loop(0, nchunks, chunk, 0)

    return body(ktab, vtab, idx_flat)


def _attn_body(q_ref, knb_ref, vnb_ref, wot_ref, bo_ref, o_ref):
    q = q_ref[...]
    cols = [[None] * K for _ in range(H)]
    for k in range(K):
        t = q * knb_ref[:, k * D:(k + 1) * D]
        for h in range(H):
            cols[h][k] = jnp.sum(t[:, h * DH:(h + 1) * DH], axis=1,
                                 keepdims=True)
    heads = []
    for h in range(H):
        s = jnp.concatenate(cols[h], axis=1) * SCALE
        m = jnp.max(s, axis=1, keepdims=True)
        e = jnp.exp(s - m)
        w = e / jnp.sum(e, axis=1, keepdims=True)
        acc = jnp.zeros((q.shape[0], DH), jnp.float32)
        for k in range(K):
            acc = acc + (vnb_ref[:, k * D + h * DH:k * D + (h + 1) * DH]
                         * w[:, k:k + 1])
        heads.append(acc)
    att = jnp.concatenate(heads, axis=1)
    o_ref[...] = (jnp.dot(att, wot_ref[...],
                          preferred_element_type=jnp.float32) + bo_ref[0:1, :])


def _attn(qp, knb, vnb, wot, bo8):
    return pl.pallas_call(
        _attn_body,
        grid=(N1 // BN,),
        in_specs=[
            pl.BlockSpec((BN, D), lambda i: (i, 0)),
            pl.BlockSpec((BN, K * D), lambda i: (i, 0)),
            pl.BlockSpec((BN, K * D), lambda i: (i, 0)),
            pl.BlockSpec((D, D), lambda i: (0, 0)),
            pl.BlockSpec((8, D), lambda i: (0, 0)),
        ],
        out_specs=pl.BlockSpec((BN, D), lambda i: (i, 0)),
        out_shape=jax.ShapeDtypeStruct((N1, D), jnp.float32),
    )(qp, knb, vnb, wot, bo8)


def kernel(query_features, key_features, query_positions, key_positions,
           Wq, bq, Wk, bk, Wv, bv, Wo, bo):
    q2 = jnp.sum(query_positions * query_positions, axis=1, keepdims=True)
    k2 = jnp.sum(key_positions * key_positions, axis=1, keepdims=True)
    qp_pad = jnp.concatenate(
        [query_positions, q2, jnp.zeros((N1, 4), jnp.float32)], axis=1)
    kpt = key_positions.T
    kpt_pad = jnp.concatenate(
        [jnp.concatenate([kpt, jnp.zeros((1, N2), jnp.float32), k2.T,
                          jnp.zeros((3, N2), jnp.float32)], axis=0),
         jnp.zeros((8, NPAD - N2), jnp.float32)], axis=1)

    knn_idx = _knn(qp_pad, kpt_pad)

    b8 = lambda b: jnp.broadcast_to(b[None, :], (8, D))
    qproj = _proj(query_features, Wq.T, b8(bq))
    kproj, vproj = _proj2(key_features, Wk.T, b8(bk), Wv.T, b8(bv))

    knb, vnb = _gather(kproj, vproj, knn_idx.reshape(-1))

    return _attn(qproj, knb.reshape(N1, K * D), vnb.reshape(N1, K * D),
                 Wo.T, b8(bo))


# speedup vs baseline: 2.8369x; 2.4955x over previous
import functools

import jax
import jax.numpy as jnp
from jax import lax
from jax.experimental import pallas as pl
from jax.experimental.pallas import tpu as pltpu
from jax.experimental.pallas import tpu_sc as plsc

N1 = 10000
N2 = 10000
NPAD = 10240
D = 512
H = 8
DH = 64
K = 16
SCALE = DH ** -0.5

BQ = 200
CH = 128
BN = 80
BP = 1000


def _knn_body(qp_ref, kpt_ref, idx_ref, d2_ref):
    qp = qp_ref[...]
    kpt = kpt_ref[...]
    q2 = qp[:, 3:4]
    k2 = kpt[4:5, :]
    qk = lax.dot_general(qp, kpt, (((1,), (0,)), ((), ())),
                         preferred_element_type=jnp.float32)
    col = lax.broadcasted_iota(jnp.int32, (1, NPAD), 1)
    dist = jnp.sqrt(jnp.maximum((q2 + k2) - 2.0 * qk, 0.0))
    d2_ref[...] = jnp.where(col < N2, dist, jnp.inf)

    col2 = lax.broadcasted_iota(jnp.int32, (BQ, NPAD), 1)
    outs = []
    for _ in range(K):
        d = d2_ref[...]
        m = jnp.min(d, axis=1, keepdims=True)
        amin = jnp.min(jnp.where(d == m, col2, NPAD), axis=1,
                       keepdims=True)
        d2_ref[...] = jnp.where(col2 == amin, jnp.inf, d)
        outs.append(amin)
    idx_ref[...] = jnp.concatenate(outs, axis=1)


def _knn(qp_pad, kpt_pad):
    return pl.pallas_call(
        _knn_body,
        grid=(N1 // BQ,),
        in_specs=[
            pl.BlockSpec((BQ, 8), lambda i: (i, 0)),
            pl.BlockSpec((8, NPAD), lambda i: (0, 0)),
        ],
        out_specs=pl.BlockSpec((BQ, K), lambda i: (i, 0)),
        out_shape=jax.ShapeDtypeStruct((N1, K), jnp.int32),
        scratch_shapes=[pltpu.VMEM((BQ, NPAD), jnp.float32)],
    )(qp_pad, kpt_pad)


def _proj_body(x_ref, wt_ref, b_ref, o_ref):
    o_ref[...] = (jnp.dot(x_ref[...], wt_ref[...],
                          preferred_element_type=jnp.float32)
                  + b_ref[0:1, :])


def _proj(x, wt, b8):
    n = x.shape[0]
    return pl.pallas_call(
        _proj_body,
        grid=(n // BP,),
        in_specs=[
            pl.BlockSpec((BP, D), lambda i: (i, 0)),
            pl.BlockSpec((D, D), lambda i: (0, 0)),
            pl.BlockSpec((8, D), lambda i: (0, 0)),
        ],
        out_specs=pl.BlockSpec((BP, D), lambda i: (i, 0)),
        out_shape=jax.ShapeDtypeStruct((n, D), jnp.float32),
    )(x, wt, b8)


def _proj2_body(x_ref, wt1_ref, b1_ref, wt2_ref, b2_ref, o1_ref, o2_ref):
    x = x_ref[...]
    o1_ref[...] = (jnp.dot(x, wt1_ref[...],
                           preferred_element_type=jnp.float32) + b1_ref[0:1, :])
    o2_ref[...] = (jnp.dot(x, wt2_ref[...],
                           preferred_element_type=jnp.float32) + b2_ref[0:1, :])


def _proj2(x, wt1, b18, wt2, b28):
    n = x.shape[0]
    return pl.pallas_call(
        _proj2_body,
        grid=(n // BP,),
        in_specs=[
            pl.BlockSpec((BP, D), lambda i: (i, 0)),
            pl.BlockSpec((D, D), lambda i: (0, 0)),
            pl.BlockSpec((8, D), lambda i: (0, 0)),
            pl.BlockSpec((D, D), lambda i: (0, 0)),
            pl.BlockSpec((8, D), lambda i: (0, 0)),
        ],
        out_specs=[
            pl.BlockSpec((BP, D), lambda i: (i, 0)),
            pl.BlockSpec((BP, D), lambda i: (i, 0)),
        ],
        out_shape=[
            jax.ShapeDtypeStruct((n, D), jnp.float32),
            jax.ShapeDtypeStruct((n, D), jnp.float32),
        ],
    )(x, wt1, b18, wt2, b28)


GC = 40


def _gather(ktab, vtab, idx_flat):
    info = plsc.get_sparse_core_info()
    nw = info.num_cores * info.num_subcores
    b_per_w = (N1 * K) // nw
    nchunks = b_per_w // GC
    mesh = plsc.VectorSubcoreMesh(core_axis_name="c", subcore_axis_name="s")

    @functools.partial(
        pl.kernel, mesh=mesh,
        out_type=[
            jax.ShapeDtypeStruct((N1 * K, D), jnp.float32),
            jax.ShapeDtypeStruct((N1 * K, D), jnp.float32),
        ],
        scratch_types=[
            pltpu.VMEM((b_per_w,), jnp.int32),
            pltpu.VMEM((GC, D), jnp.float32),
            pltpu.VMEM((GC, D), jnp.float32),
            pltpu.SemaphoreType.DMA,
            pltpu.SemaphoreType.DMA,
        ],
    )
    def body(k_hbm, v_hbm, idx_hbm, knb_hbm, vnb_hbm,
             idx_v, krows, vrows, ksem, vsem):
        wid = lax.axis_index("s") * info.num_cores + lax.axis_index("c")
        base = wid * b_per_w
        pltpu.sync_copy(idx_hbm.at[pl.ds(base, b_per_w)], idx_v)

        def chunk(j, carry):
            off = j * GC
            isl = idx_v.at[pl.ds(off, GC)]
            pltpu.async_copy(k_hbm.at[isl], krows, ksem).wait()
            pltpu.async_copy(v_hbm.at[isl], vrows, vsem).wait()
            pltpu.sync_copy(krows, knb_hbm.at[pl.ds(base + off, GC)])
            pltpu.sync_copy(vrows, vnb_hbm.at[pl.ds(base + off, GC)])
            return carry

        lax.fori_loop(0, nchunks, chunk, 0)

    return body(ktab, vtab, idx_flat)


def _attn_body(q_ref, knb_ref, vnb_ref, wot_ref, bo_ref, o_ref):
    q = q_ref[...]
    cols = [[None] * K for _ in range(H)]
    for k in range(K):
        t = q * knb_ref[:, k * D:(k + 1) * D]
        for h in range(H):
            cols[h][k] = jnp.sum(t[:, h * DH:(h + 1) * DH], axis=1,
                                 keepdims=True)
    heads = []
    for h in range(H):
        s = jnp.concatenate(cols[h], axis=1) * SCALE
        m = jnp.max(s, axis=1, keepdims=True)
        e = jnp.exp(s - m)
        w = e / jnp.sum(e, axis=1, keepdims=True)
        acc = jnp.zeros((q.shape[0], DH), jnp.float32)
        for k in range(K):
            acc = acc + (vnb_ref[:, k * D + h * DH:k * D + (h + 1) * DH]
                         * w[:, k:k + 1])
        heads.append(acc)
    att = jnp.concatenate(heads, axis=1)
    o_ref[...] = (jnp.dot(att, wot_ref[...],
                          preferred_element_type=jnp.float32) + bo_ref[0:1, :])


def _attn(qp, knb, vnb, wot, bo8):
    return pl.pallas_call(
        _attn_body,
        grid=(N1 // BN,),
        in_specs=[
            pl.BlockSpec((BN, D), lambda i: (i, 0)),
            pl.BlockSpec((BN, K * D), lambda i: (i, 0)),
            pl.BlockSpec((BN, K * D), lambda i: (i, 0)),
            pl.BlockSpec((D, D), lambda i: (0, 0)),
            pl.BlockSpec((8, D), lambda i: (0, 0)),
        ],
        out_specs=pl.BlockSpec((BN, D), lambda i: (i, 0)),
        out_shape=jax.ShapeDtypeStruct((N1, D), jnp.float32),
    )(qp, knb, vnb, wot, bo8)


def kernel(query_features, key_features, query_positions, key_positions,
           Wq, bq, Wk, bk, Wv, bv, Wo, bo):
    q2 = jnp.sum(query_positions * query_positions, axis=1, keepdims=True)
    k2 = jnp.sum(key_positions * key_positions, axis=1, keepdims=True)
    qp_pad = jnp.concatenate(
        [query_positions, q2, jnp.zeros((N1, 4), jnp.float32)], axis=1)
    kpt = key_positions.T
    kpt_pad = jnp.concatenate(
        [jnp.concatenate([kpt, jnp.zeros((1, N2), jnp.float32), k2.T,
                          jnp.zeros((3, N2), jnp.float32)], axis=0),
         jnp.zeros((8, NPAD - N2), jnp.float32)], axis=1)

    knn_idx = _knn(qp_pad, kpt_pad)

    b8 = lambda b: jnp.broadcast_to(b[None, :], (8, D))
    qproj = _proj(query_features, Wq.T, b8(bq))
    kproj, vproj = _proj2(key_features, Wk.T, b8(bk), Wv.T, b8(bv))

    knb, vnb = _gather(kproj, vproj, knn_idx.reshape(-1))

    return _attn(qproj, knb.reshape(N1, K * D), vnb.reshape(N1, K * D),
                 Wo.T, b8(bo))


# speedup vs baseline: 3.0193x; 1.0643x over previous
import functools

import jax
import jax.numpy as jnp
from jax import lax
from jax.experimental import pallas as pl
from jax.experimental.pallas import tpu as pltpu
from jax.experimental.pallas import tpu_sc as plsc

N1 = 10000
N2 = 10000
NPAD = 10240
D = 512
H = 8
DH = 64
K = 16
SCALE = DH ** -0.5

BQ = 200
CH = 128
NG = NPAD // CH
R1 = 8
BN = 80
BP = 1000


def _knn_body(qp_ref, kpt_ref, idx_ref, d2_ref):
    qp = qp_ref[...]
    kpt = kpt_ref[...]
    q2 = qp[:, 3:4]
    k2 = kpt[4:5, :]
    qk = lax.dot_general(qp, kpt, (((1,), (0,)), ((), ())),
                         preferred_element_type=jnp.float32)
    col = lax.broadcasted_iota(jnp.int32, (1, NPAD), 1)
    dist = jnp.sqrt(jnp.maximum((q2 + k2) - 2.0 * qk, 0.0))

    d2_ref[...] = jnp.where(col < N2, dist, jnp.inf)

    colf = lax.broadcasted_iota(jnp.int32, (BQ, NPAD), 1).astype(jnp.float32)
    outs = []
    for _ in range(K):
        d = d2_ref[...]
        m = jnp.min(d, axis=1, keepdims=True)
        amin = jnp.min(jnp.where(d == m, colf, float(NPAD)), axis=1,
                       keepdims=True)
        d2_ref[...] = jnp.where(colf == amin, jnp.inf, d)
        outs.append(amin.astype(jnp.int32))
    idx_ref[...] = jnp.concatenate(outs, axis=1)


def _knn(qp_pad, kpt_pad):
    return pl.pallas_call(
        _knn_body,
        grid=(N1 // BQ,),
        in_specs=[
            pl.BlockSpec((BQ, 8), lambda i: (i, 0)),
            pl.BlockSpec((8, NPAD), lambda i: (0, 0)),
        ],
        out_specs=pl.BlockSpec((BQ, K), lambda i: (i, 0)),
        out_shape=jax.ShapeDtypeStruct((N1, K), jnp.int32),
        scratch_shapes=[pltpu.VMEM((BQ, NPAD), jnp.float32)],
    )(qp_pad, kpt_pad)


def _proj_body(x_ref, wt_ref, b_ref, o_ref):
    o_ref[...] = (jnp.dot(x_ref[...], wt_ref[...],
                          preferred_element_type=jnp.float32)
                  + b_ref[0:1, :])


def _proj(x, wt, b8):
    n = x.shape[0]
    return pl.pallas_call(
        _proj_body,
        grid=(n // BP,),
        in_specs=[
            pl.BlockSpec((BP, D), lambda i: (i, 0)),
            pl.BlockSpec((D, D), lambda i: (0, 0)),
            pl.BlockSpec((8, D), lambda i: (0, 0)),
        ],
        out_specs=pl.BlockSpec((BP, D), lambda i: (i, 0)),
        out_shape=jax.ShapeDtypeStruct((n, D), jnp.float32),
    )(x, wt, b8)


def _proj2_body(x_ref, wt1_ref, b1_ref, wt2_ref, b2_ref, o1_ref, o2_ref):
    x = x_ref[...]
    o1_ref[...] = (jnp.dot(x, wt1_ref[...],
                           preferred_element_type=jnp.float32) + b1_ref[0:1, :])
    o2_ref[...] = (jnp.dot(x, wt2_ref[...],
                           preferred_element_type=jnp.float32) + b2_ref[0:1, :])


def _proj2(x, wt1, b18, wt2, b28):
    n = x.shape[0]
    return pl.pallas_call(
        _proj2_body,
        grid=(n // BP,),
        in_specs=[
            pl.BlockSpec((BP, D), lambda i: (i, 0)),
            pl.BlockSpec((D, D), lambda i: (0, 0)),
            pl.BlockSpec((8, D), lambda i: (0, 0)),
            pl.BlockSpec((D, D), lambda i: (0, 0)),
            pl.BlockSpec((8, D), lambda i: (0, 0)),
        ],
        out_specs=[
            pl.BlockSpec((BP, D), lambda i: (i, 0)),
            pl.BlockSpec((BP, D), lambda i: (i, 0)),
        ],
        out_shape=[
            jax.ShapeDtypeStruct((n, D), jnp.float32),
            jax.ShapeDtypeStruct((n, D), jnp.float32),
        ],
    )(x, wt1, b18, wt2, b28)


GC = 40


def _gather(ktab, vtab, idx_flat):
    info = plsc.get_sparse_core_info()
    nw = info.num_cores * info.num_subcores
    b_per_w = (N1 * K) // nw
    nchunks = b_per_w // GC
    mesh = plsc.VectorSubcoreMesh(core_axis_name="c", subcore_axis_name="s")

    @functools.partial(
        pl.kernel, mesh=mesh,
        out_type=[
            jax.ShapeDtypeStruct((N1 * K, D), jnp.float32),
            jax.ShapeDtypeStruct((N1 * K, D), jnp.float32),
        ],
        scratch_types=[
            pltpu.VMEM((b_per_w,), jnp.int32),
            pltpu.VMEM((GC, D), jnp.float32),
            pltpu.VMEM((GC, D), jnp.float32),
            pltpu.SemaphoreType.DMA,
            pltpu.SemaphoreType.DMA,
        ],
    )
    def body(k_hbm, v_hbm, idx_hbm, knb_hbm, vnb_hbm,
             idx_v, krows, vrows, ksem, vsem):
        wid = lax.axis_index("s") * info.num_cores + lax.axis_index("c")
        base = wid * b_per_w
        pltpu.sync_copy(idx_hbm.at[pl.ds(base, b_per_w)], idx_v)

        def chunk(j, carry):
            off = j * GC
            isl = idx_v.at[pl.ds(off, GC)]
            pltpu.async_copy(k_hbm.at[isl], krows, ksem).wait()
            pltpu.async_copy(v_hbm.at[isl], vrows, vsem).wait()
            pltpu.sync_copy(krows, knb_hbm.at[pl.ds(base + off, GC)])
            pltpu.sync_copy(vrows, vnb_hbm.at[pl.ds(base + off, GC)])
            return carry

        lax.fori_loop(0, nchunks, chunk, 0)

    return body(ktab, vtab, idx_flat)


def _attn_body(q_ref, knb_ref, vnb_ref, wot_ref, bo_ref, o_ref):
    q = q_ref[...]
    cols = [[None] * K for _ in range(H)]
    for k in range(K):
        t = q * knb_ref[:, k * D:(k + 1) * D]
        for h in range(H):
            cols[h][k] = jnp.sum(t[:, h * DH:(h + 1) * DH], axis=1,
                                 keepdims=True)
    heads = []
    for h in range(H):
        s = jnp.concatenate(cols[h], axis=1) * SCALE
        m = jnp.max(s, axis=1, keepdims=True)
        e = jnp.exp(s - m)
        w = e / jnp.sum(e, axis=1, keepdims=True)
        acc = jnp.zeros((q.shape[0], DH), jnp.float32)
        for k in range(K):
            acc = acc + (vnb_ref[:, k * D + h * DH:k * D + (h + 1) * DH]
                         * w[:, k:k + 1])
        heads.append(acc)
    att = jnp.concatenate(heads, axis=1)
    o_ref[...] = (jnp.dot(att, wot_ref[...],
                          preferred_element_type=jnp.float32) + bo_ref[0:1, :])


def _attn(qp, knb, vnb, wot, bo8):
    return pl.pallas_call(
        _attn_body,
        grid=(N1 // BN,),
        in_specs=[
            pl.BlockSpec((BN, D), lambda i: (i, 0)),
            pl.BlockSpec((BN, K * D), lambda i: (i, 0)),
            pl.BlockSpec((BN, K * D), lambda i: (i, 0)),
            pl.BlockSpec((D, D), lambda i: (0, 0)),
            pl.BlockSpec((8, D), lambda i: (0, 0)),
        ],
        out_specs=pl.BlockSpec((BN, D), lambda i: (i, 0)),
        out_shape=jax.ShapeDtypeStruct((N1, D), jnp.float32),
    )(qp, knb, vnb, wot, bo8)


def kernel(query_features, key_features, query_positions, key_positions,
           Wq, bq, Wk, bk, Wv, bv, Wo, bo):
    q2 = jnp.sum(query_positions * query_positions, axis=1, keepdims=True)
    k2 = jnp.sum(key_positions * key_positions, axis=1, keepdims=True)
    qp_pad = jnp.concatenate(
        [query_positions, q2, jnp.zeros((N1, 4), jnp.float32)], axis=1)
    kpt = key_positions.T
    kpt_pad = jnp.concatenate(
        [jnp.concatenate([kpt, jnp.zeros((1, N2), jnp.float32), k2.T,
                          jnp.zeros((3, N2), jnp.float32)], axis=0),
         jnp.zeros((8, NPAD - N2), jnp.float32)], axis=1)

    knn_idx = _knn(qp_pad, kpt_pad)

    b8 = lambda b: jnp.broadcast_to(b[None, :], (8, D))
    qproj = _proj(query_features, Wq.T, b8(bq))
    kproj, vproj = _proj2(key_features, Wk.T, b8(bk), Wv.T, b8(bv))

    knb, vnb = _gather(kproj, vproj, knn_idx.reshape(-1))

    return _attn(qproj, knb.reshape(N1, K * D), vnb.reshape(N1, K * D),
                 Wo.T, b8(bo))
